# baseline (speedup 1.0000x reference)
import functools

import jax
import jax.numpy as jnp
from jax import lax
from jax.experimental import pallas as pl
from jax.experimental.pallas import tpu as pltpu
from jax.experimental.pallas import tpu_sc as plsc

CH = 14
CW = 14
H = 64
W = 64
C = 256
PATCH = 13
PROW = PATCH * C
NPIX = CH * CW
NCHUNK = 13
PADPIX = NCHUNK * 16
OSTR = NPIX + 1

_NC = 2
_NS = 16
NTILES = _NC * _NS
M = 1000
MPAD = 1024
BPT = MPAD // NTILES


def _floor(x):
    t = x.astype(jnp.int32).astype(jnp.float32)
    return t - jnp.where(t > x, jnp.float32(1.0), jnp.float32(0.0))


def _ceil(x):
    t = x.astype(jnp.int32).astype(jnp.float32)
    return t + jnp.where(t < x, jnp.float32(1.0), jnp.float32(0.0))


def _roialign_sc(fm_flat, boxes_flat, bind, iofp, jofp):
    mesh = plsc.VectorSubcoreMesh(core_axis_name="c", subcore_axis_name="s",
                                  num_cores=_NC, num_subcores=_NS)

    @functools.partial(
        pl.kernel,
        out_type=jax.ShapeDtypeStruct((M, C, CH, CW), jnp.float32),
        mesh=mesh,
        compiler_params=pltpu.CompilerParams(needs_layout_passes=False,
                                             use_tc_tiling_on_sc=False),
        scratch_types=[
            pltpu.VMEM((BPT * 4 + 16,), jnp.float32),
            pltpu.VMEM((BPT + 16,), jnp.int32),
            pltpu.VMEM((PADPIX,), jnp.float32),
            pltpu.VMEM((PADPIX,), jnp.float32),
            pltpu.VMEM((PATCH * PROW,), jnp.float32),
            pltpu.VMEM((C, CH, CW), jnp.float32),
            pltpu.SemaphoreType.DMA,
        ],
    )
    def k(fm_hbm, boxes_hbm, bind_hbm, iofp_hbm, jofp_hbm, out_hbm,
          boxes_v, bind_v, iofp_v, jofp_v, patch_v, out_v, dsem):
        wid = lax.axis_index("s") * _NC + lax.axis_index("c")
        m0 = wid * BPT
        count = jnp.minimum(BPT, M - m0)
        pltpu.sync_copy(boxes_hbm.at[pl.ds(pl.multiple_of(m0 * 4, BPT * 4),
                                           BPT * 4)],
                        boxes_v.at[pl.ds(0, BPT * 4)])
        pltpu.sync_copy(bind_hbm.at[pl.ds(pl.multiple_of(m0, BPT), BPT)],
                        bind_v.at[pl.ds(0, BPT)])
        pltpu.sync_copy(iofp_hbm, iofp_v)
        pltpu.sync_copy(jofp_hbm, jofp_v)
        iota = lax.broadcasted_iota(jnp.int32, (16,), 0)

        @pl.loop(0, count)
        def _box(li):
            bv = boxes_v[pl.ds(4 * li, 16)]
            x1 = bv[0]
            y1 = bv[1]
            x2 = bv[2]
            y2 = bv[3]
            b = bind_v[pl.ds(li, 16)][0]
            spw = (x2 - x1) * jnp.float32(1.0 / CW)
            sph = (y2 - y1) * jnp.float32(1.0 / CH)
            nx0 = (x1 + spw * jnp.float32(0.5) - jnp.float32(0.5)) * jnp.float32(1.0 / (W - 1))
            ny0 = (y1 + sph * jnp.float32(0.5) - jnp.float32(0.5)) * jnp.float32(1.0 / (H - 1))
            nwd = spw * jnp.float32(CW - 1) * jnp.float32(1.0 / (W - 1))
            nht = sph * jnp.float32(CH - 1) * jnp.float32(1.0 / (H - 1))
            basex = nx0 * jnp.float32(W - 1)
            stepx = nwd * jnp.float32(W - 1) * jnp.float32(1.0 / (CW - 1))
            basey = ny0 * jnp.float32(H - 1)
            stepy = nht * jnp.float32(H - 1) * jnp.float32(1.0 / (CH - 1))
            x0 = jnp.clip(_floor(basex).astype(jnp.int32), 0, W - PATCH)
            y0 = jnp.clip(_floor(basey).astype(jnp.int32), 0, H - PATCH)

            rowbase = ((b * H + y0) * W + x0) * C
            cps = []
            for dy in range(PATCH):
                cps.append(pltpu.async_copy(
                    fm_hbm.at[pl.ds(pl.multiple_of(rowbase + dy * (W * C), C),
                                    PROW)],
                    patch_v.at[pl.ds(dy * PROW, PROW)], dsem))
            for cp in cps:
                cp.wait()

            for kk in range(NCHUNK):
                iyf = iofp_v[pl.ds(kk * 16, 16)]
                jxf = jofp_v[pl.ds(kk * 16, 16)]
                in_y = basey + iyf * stepy
                in_x = basex + jxf * stepx
                vyv = jnp.where((in_y >= 0.0) & (in_y <= jnp.float32(H - 1)),
                                jnp.float32(1.0), jnp.float32(0.0))
                vxv = jnp.where((in_x >= 0.0) & (in_x <= jnp.float32(W - 1)),
                                jnp.float32(1.0), jnp.float32(0.0))
                ylo_f = _floor(in_y)
                xlo_f = _floor(in_x)
                yhi_f = _ceil(in_y)
                xhi_f = _ceil(in_x)
                ylerp = in_y - ylo_f
                xlerp = in_x - xlo_f
                pyl = jnp.clip(
                    jnp.clip(ylo_f, 0.0, jnp.float32(H - 1)).astype(jnp.int32)
                    - y0, 0, PATCH - 1)
                pyh = jnp.clip(
                    jnp.clip(yhi_f, 0.0, jnp.float32(H - 1)).astype(jnp.int32)
                    - y0, 0, PATCH - 1)
                pxl = jnp.clip(
                    jnp.clip(xlo_f, 0.0, jnp.float32(W - 1)).astype(jnp.int32)
                    - x0, 0, PATCH - 1)
                pxh = jnp.clip(
                    jnp.clip(xhi_f, 0.0, jnp.float32(W - 1)).astype(jnp.int32)
                    - x0, 0, PATCH - 1)
                vv = vyv * vxv
                omy = jnp.float32(1.0) - ylerp
                omx = jnp.float32(1.0) - xlerp
                w1 = vv * omy * omx
                w2 = vv * omy * xlerp
                w3 = vv * ylerp * omx
                w4 = vv * ylerp * xlerp
                b1 = (pyl * PATCH + pxl) * C
                b2 = (pyl * PATCH + pxh) * C
                b3 = (pyh * PATCH + pxl) * C
                b4 = (pyh * PATCH + pxh) * C
                pv = iota + (kk * 16)
                iv = iyf.astype(jnp.int32)
                jv = jxf.astype(jnp.int32)

                @plsc.parallel_loop(0, C, unroll=4,
                                    carry=(iota, b1, b2, b3, b4, pv, iv, jv,
                                           w1, w2, w3, w4))
                def _chan(cc, carry):
                    ci, a1, a2, a3, a4, pv_, iv_, jv_, q1, q2, q3, q4 = carry
                    tl = plsc.load_gather(patch_v, [a1 + ci])
                    tr = plsc.load_gather(patch_v, [a2 + ci])
                    bl = plsc.load_gather(patch_v, [a3 + ci])
                    br = plsc.load_gather(patch_v, [a4 + ci])
                    acc = q1 * tl + q2 * tr + q3 * bl + q4 * br
                    plsc.store_scatter(out_v, [ci, iv_, jv_], acc,
                                       mask=pv_ < NPIX)
                    ci2 = (ci + 1) & (C - 1)
                    return (ci2, a1, a2, a3, a4, pv_, iv_, jv_,
                            q1, q2, q3, q4)

            pltpu.sync_copy(out_v, out_hbm.at[m0 + li])

    return k(fm_flat, boxes_flat, bind, iofp, jofp)


def kernel(featuremap, boxes, box_ind):
    fm_flat = jnp.transpose(featuremap, (0, 2, 3, 1)).reshape(-1)
    boxes_flat = jnp.pad(boxes, ((0, MPAD - M), (0, 0))).reshape(-1)
    bind = jnp.pad(box_ind, (0, MPAD - M))
    p = jnp.arange(PADPIX, dtype=jnp.int32)
    pp = jnp.where(p < NPIX, p, 0)
    iofp = (pp // CW).astype(jnp.float32)
    jofp = (pp % CW).astype(jnp.float32)
    return _roialign_sc(fm_flat, boxes_flat, bind, iofp, jofp)


# speedup vs baseline: 2.0284x; 2.0284x over previous
import functools

import jax
import jax.numpy as jnp
from jax import lax
from jax.experimental import pallas as pl
from jax.experimental.pallas import tpu as pltpu
from jax.experimental.pallas import tpu_sc as plsc

CH = 14
CW = 14
H = 64
W = 64
C = 256
PATCH = 13
PROW = PATCH * C
NPIX = CH * CW
NCHUNK = 13
PADPIX = NCHUNK * 16
OSTR = NPIX + 1

_NC = 2
_NS = 16
NTILES = _NC * _NS
M = 1000
MPAD = 1024
BPT = MPAD // NTILES


def _floor(x):
    t = x.astype(jnp.int32).astype(jnp.float32)
    return t - jnp.where(t > x, jnp.float32(1.0), jnp.float32(0.0))


def _ceil(x):
    t = x.astype(jnp.int32).astype(jnp.float32)
    return t + jnp.where(t < x, jnp.float32(1.0), jnp.float32(0.0))


def _roialign_sc(fm_flat, boxes_flat, bind, iofp, jofp):
    mesh = plsc.VectorSubcoreMesh(core_axis_name="c", subcore_axis_name="s",
                                  num_cores=_NC, num_subcores=_NS)

    @functools.partial(
        pl.kernel,
        out_type=jax.ShapeDtypeStruct((M, C, NPIX), jnp.float32),
        mesh=mesh,
        compiler_params=pltpu.CompilerParams(needs_layout_passes=False,
                                             use_tc_tiling_on_sc=False),
        scratch_types=[
            pltpu.VMEM((BPT * 4 + 16,), jnp.float32),
            pltpu.VMEM((BPT + 16,), jnp.int32),
            pltpu.VMEM((PADPIX,), jnp.float32),
            pltpu.VMEM((PADPIX,), jnp.float32),
            pltpu.VMEM((PATCH * PROW,), jnp.float32),
            pltpu.VMEM((C, NPIX), jnp.float32),
            pltpu.SemaphoreType.DMA,
        ],
    )
    def k(fm_hbm, boxes_hbm, bind_hbm, iofp_hbm, jofp_hbm, out_hbm,
          boxes_v, bind_v, iofp_v, jofp_v, patch_v, out_v, dsem):
        wid = lax.axis_index("s") * _NC + lax.axis_index("c")
        m0 = wid * BPT
        count = jnp.minimum(BPT, M - m0)
        pltpu.sync_copy(boxes_hbm.at[pl.ds(pl.multiple_of(m0 * 4, BPT * 4),
                                           BPT * 4)],
                        boxes_v.at[pl.ds(0, BPT * 4)])
        pltpu.sync_copy(bind_hbm.at[pl.ds(pl.multiple_of(m0, BPT), BPT)],
                        bind_v.at[pl.ds(0, BPT)])
        pltpu.sync_copy(iofp_hbm, iofp_v)
        pltpu.sync_copy(jofp_hbm, jofp_v)
        iota = lax.broadcasted_iota(jnp.int32, (16,), 0)

        @pl.loop(0, count)
        def _box(li):
            bv = boxes_v[pl.ds(4 * li, 16)]
            x1 = bv[0]
            y1 = bv[1]
            x2 = bv[2]
            y2 = bv[3]
            b = bind_v[pl.ds(li, 16)][0]
            spw = (x2 - x1) * jnp.float32(1.0 / CW)
            sph = (y2 - y1) * jnp.float32(1.0 / CH)
            nx0 = (x1 + spw * jnp.float32(0.5) - jnp.float32(0.5)) * jnp.float32(1.0 / (W - 1))
            ny0 = (y1 + sph * jnp.float32(0.5) - jnp.float32(0.5)) * jnp.float32(1.0 / (H - 1))
            nwd = spw * jnp.float32(CW - 1) * jnp.float32(1.0 / (W - 1))
            nht = sph * jnp.float32(CH - 1) * jnp.float32(1.0 / (H - 1))
            basex = nx0 * jnp.float32(W - 1)
            stepx = nwd * jnp.float32(W - 1) * jnp.float32(1.0 / (CW - 1))
            basey = ny0 * jnp.float32(H - 1)
            stepy = nht * jnp.float32(H - 1) * jnp.float32(1.0 / (CH - 1))
            x0 = jnp.clip(_floor(basex).astype(jnp.int32), 0, W - PATCH)
            y0 = jnp.clip(_floor(basey).astype(jnp.int32), 0, H - PATCH)

            rowbase = ((b * H + y0) * W + x0) * C
            cps = []
            for dy in range(PATCH):
                cps.append(pltpu.async_copy(
                    fm_hbm.at[pl.ds(pl.multiple_of(rowbase + dy * (W * C), C),
                                    PROW)],
                    patch_v.at[pl.ds(dy * PROW, PROW)], dsem))
            for cp in cps:
                cp.wait()

            for kk in range(NCHUNK):
                iyf = iofp_v[pl.ds(kk * 16, 16)]
                jxf = jofp_v[pl.ds(kk * 16, 16)]
                in_y = basey + iyf * stepy
                in_x = basex + jxf * stepx
                vyv = jnp.where((in_y >= 0.0) & (in_y <= jnp.float32(H - 1)),
                                jnp.float32(1.0), jnp.float32(0.0))
                vxv = jnp.where((in_x >= 0.0) & (in_x <= jnp.float32(W - 1)),
                                jnp.float32(1.0), jnp.float32(0.0))
                ylo_f = _floor(in_y)
                xlo_f = _floor(in_x)
                yhi_f = _ceil(in_y)
                xhi_f = _ceil(in_x)
                ylerp = in_y - ylo_f
                xlerp = in_x - xlo_f
                pyl = jnp.clip(
                    jnp.clip(ylo_f, 0.0, jnp.float32(H - 1)).astype(jnp.int32)
                    - y0, 0, PATCH - 1)
                pyh = jnp.clip(
                    jnp.clip(yhi_f, 0.0, jnp.float32(H - 1)).astype(jnp.int32)
                    - y0, 0, PATCH - 1)
                pxl = jnp.clip(
                    jnp.clip(xlo_f, 0.0, jnp.float32(W - 1)).astype(jnp.int32)
                    - x0, 0, PATCH - 1)
                pxh = jnp.clip(
                    jnp.clip(xhi_f, 0.0, jnp.float32(W - 1)).astype(jnp.int32)
                    - x0, 0, PATCH - 1)
                vv = vyv * vxv
                omy = jnp.float32(1.0) - ylerp
                omx = jnp.float32(1.0) - xlerp
                w1 = vv * omy * omx
                w2 = vv * omy * xlerp
                w3 = vv * ylerp * omx
                w4 = vv * ylerp * xlerp
                b1 = (pyl * PATCH + pxl) * C
                b2 = (pyl * PATCH + pxh) * C
                b3 = (pyh * PATCH + pxl) * C
                b4 = (pyh * PATCH + pxh) * C
                pv = iota + (kk * 16)

                @plsc.parallel_loop(0, C, unroll=4,
                                    carry=(iota, b1, b2, b3, b4, pv,
                                           w1, w2, w3, w4))
                def _chan(cc, carry):
                    ci, a1, a2, a3, a4, pv_, q1, q2, q3, q4 = carry
                    tl = plsc.load_gather(patch_v, [a1 + ci])
                    tr = plsc.load_gather(patch_v, [a2 + ci])
                    bl = plsc.load_gather(patch_v, [a3 + ci])
                    br = plsc.load_gather(patch_v, [a4 + ci])
                    acc = q1 * tl + q2 * tr + q3 * bl + q4 * br
                    plsc.store_scatter(out_v, [ci, pv_], acc,
                                       mask=pv_ < NPIX)
                    ci2 = (ci + 1) & (C - 1)
                    return (ci2, a1, a2, a3, a4, pv_, q1, q2, q3, q4)

            pltpu.sync_copy(out_v, out_hbm.at[m0 + li])

    return k(fm_flat, boxes_flat, bind, iofp, jofp)


def kernel(featuremap, boxes, box_ind):
    fm_flat = jnp.transpose(featuremap, (0, 2, 3, 1)).reshape(-1)
    boxes_flat = jnp.pad(boxes, ((0, MPAD - M), (0, 0))).reshape(-1)
    bind = jnp.pad(box_ind, (0, MPAD - M))
    p = jnp.arange(PADPIX, dtype=jnp.int32)
    pp = jnp.where(p < NPIX, p, 0)
    iofp = (pp // CW).astype(jnp.float32)
    jofp = (pp % CW).astype(jnp.float32)
    out = _roialign_sc(fm_flat, boxes_flat, bind, iofp, jofp)
    return out.reshape(M, C, CH, CW)


# speedup vs baseline: 3.8302x; 1.8882x over previous
import functools

import jax
import jax.numpy as jnp
from jax import lax
from jax.experimental import pallas as pl
from jax.experimental.pallas import tpu as pltpu
from jax.experimental.pallas import tpu_sc as plsc

CH = 14
CW = 14
H = 64
W = 64
C = 256
PATCH = 13
PROW = PATCH * C
NPIX = CH * CW
NCHUNK = 13
PADPIX = NCHUNK * 16
OSTR = NPIX + 1

_NC = 2
_NS = 16
NTILES = _NC * _NS
M = 1000
MPAD = 1024
BPT = MPAD // NTILES


def _floor(x):
    t = x.astype(jnp.int32).astype(jnp.float32)
    return t - jnp.where(t > x, jnp.float32(1.0), jnp.float32(0.0))


def _ceil(x):
    t = x.astype(jnp.int32).astype(jnp.float32)
    return t + jnp.where(t < x, jnp.float32(1.0), jnp.float32(0.0))


def _roialign_sc(fm_flat, boxes_flat, bind, iofp, jofp):
    mesh = plsc.VectorSubcoreMesh(core_axis_name="c", subcore_axis_name="s",
                                  num_cores=_NC, num_subcores=_NS)

    @functools.partial(
        pl.kernel,
        out_type=jax.ShapeDtypeStruct((NPIX, M // 8, C // 128, 8, 128),
                                      jnp.float32),
        mesh=mesh,
        compiler_params=pltpu.CompilerParams(needs_layout_passes=False,
                                             use_tc_tiling_on_sc=False),
        scratch_types=[
            pltpu.VMEM((BPT * 4 + 16,), jnp.float32),
            pltpu.VMEM((BPT + 16,), jnp.int32),
            pltpu.VMEM((PADPIX,), jnp.float32),
            pltpu.VMEM((PADPIX,), jnp.float32),
            pltpu.VMEM((PATCH * PROW,), jnp.float32),
            pltpu.VMEM((C // 128, PADPIX, 128), jnp.float32),
            pltpu.SemaphoreType.DMA,
        ],
    )
    def k(fm_hbm, boxes_hbm, bind_hbm, iofp_hbm, jofp_hbm, out_hbm,
          boxes_v, bind_v, iofp_v, jofp_v, patch_v, out_v, dsem):
        wid = lax.axis_index("s") * _NC + lax.axis_index("c")
        m0 = wid * BPT
        count = jnp.minimum(BPT, M - m0)
        pltpu.sync_copy(boxes_hbm.at[pl.ds(pl.multiple_of(m0 * 4, BPT * 4),
                                           BPT * 4)],
                        boxes_v.at[pl.ds(0, BPT * 4)])
        pltpu.sync_copy(bind_hbm.at[pl.ds(pl.multiple_of(m0, BPT), BPT)],
                        bind_v.at[pl.ds(0, BPT)])
        pltpu.sync_copy(iofp_hbm, iofp_v)
        pltpu.sync_copy(jofp_hbm, jofp_v)
        iota = lax.broadcasted_iota(jnp.int32, (16,), 0)

        @pl.loop(0, count)
        def _box(li):
            bv = boxes_v[pl.ds(4 * li, 16)]
            x1 = bv[0]
            y1 = bv[1]
            x2 = bv[2]
            y2 = bv[3]
            b = bind_v[pl.ds(li, 16)][0]
            spw = (x2 - x1) * jnp.float32(1.0 / CW)
            sph = (y2 - y1) * jnp.float32(1.0 / CH)
            nx0 = (x1 + spw * jnp.float32(0.5) - jnp.float32(0.5)) * jnp.float32(1.0 / (W - 1))
            ny0 = (y1 + sph * jnp.float32(0.5) - jnp.float32(0.5)) * jnp.float32(1.0 / (H - 1))
            nwd = spw * jnp.float32(CW - 1) * jnp.float32(1.0 / (W - 1))
            nht = sph * jnp.float32(CH - 1) * jnp.float32(1.0 / (H - 1))
            basex = nx0 * jnp.float32(W - 1)
            stepx = nwd * jnp.float32(W - 1) * jnp.float32(1.0 / (CW - 1))
            basey = ny0 * jnp.float32(H - 1)
            stepy = nht * jnp.float32(H - 1) * jnp.float32(1.0 / (CH - 1))
            x0 = jnp.clip(_floor(basex).astype(jnp.int32), 0, W - PATCH)
            y0 = jnp.clip(_floor(basey).astype(jnp.int32), 0, H - PATCH)

            rowbase = ((b * H + y0) * W + x0) * C
            cps = []
            for dy in range(PATCH):
                cps.append(pltpu.async_copy(
                    fm_hbm.at[pl.ds(pl.multiple_of(rowbase + dy * (W * C), C),
                                    PROW)],
                    patch_v.at[pl.ds(dy * PROW, PROW)], dsem))
            for cp in cps:
                cp.wait()

            for kk in range(NCHUNK):
                iyf = iofp_v[pl.ds(kk * 16, 16)]
                jxf = jofp_v[pl.ds(kk * 16, 16)]
                in_y = basey + iyf * stepy
                in_x = basex + jxf * stepx
                vyv = jnp.where((in_y >= 0.0) & (in_y <= jnp.float32(H - 1)),
                                jnp.float32(1.0), jnp.float32(0.0))
                vxv = jnp.where((in_x >= 0.0) & (in_x <= jnp.float32(W - 1)),
                                jnp.float32(1.0), jnp.float32(0.0))
                ylo_f = _floor(in_y)
                xlo_f = _floor(in_x)
                yhi_f = _ceil(in_y)
                xhi_f = _ceil(in_x)
                ylerp = in_y - ylo_f
                xlerp = in_x - xlo_f
                pyl = jnp.clip(
                    jnp.clip(ylo_f, 0.0, jnp.float32(H - 1)).astype(jnp.int32)
                    - y0, 0, PATCH - 1)
                pyh = jnp.clip(
                    jnp.clip(yhi_f, 0.0, jnp.float32(H - 1)).astype(jnp.int32)
                    - y0, 0, PATCH - 1)
                pxl = jnp.clip(
                    jnp.clip(xlo_f, 0.0, jnp.float32(W - 1)).astype(jnp.int32)
                    - x0, 0, PATCH - 1)
                pxh = jnp.clip(
                    jnp.clip(xhi_f, 0.0, jnp.float32(W - 1)).astype(jnp.int32)
                    - x0, 0, PATCH - 1)
                vv = vyv * vxv
                omy = jnp.float32(1.0) - ylerp
                omx = jnp.float32(1.0) - xlerp
                w1 = vv * omy * omx
                w2 = vv * omy * xlerp
                w3 = vv * ylerp * omx
                w4 = vv * ylerp * xlerp
                b1 = (pyl * PATCH + pxl) * C
                b2 = (pyl * PATCH + pxh) * C
                b3 = (pyh * PATCH + pxl) * C
                b4 = (pyh * PATCH + pxh) * C
                pv = iota + (kk * 16)

                @plsc.parallel_loop(0, C, unroll=4,
                                    carry=(iota, b1, b2, b3, b4, pv,
                                           w1, w2, w3, w4))
                def _chan(cc, carry):
                    ci, a1, a2, a3, a4, pv_, q1, q2, q3, q4 = carry
                    tl = plsc.load_gather(patch_v, [a1 + ci])
                    tr = plsc.load_gather(patch_v, [a2 + ci])
                    bl = plsc.load_gather(patch_v, [a3 + ci])
                    br = plsc.load_gather(patch_v, [a4 + ci])
                    acc = q1 * tl + q2 * tr + q3 * bl + q4 * br
                    plsc.store_scatter(out_v,
                                       [ci >> 7, pv_, ci & 127], acc)
                    ci2 = (ci + 1) & (C - 1)
                    return (ci2, a1, a2, a3, a4, pv_, q1, q2, q3, q4)

            m = m0 + li
            m8 = m >> 3
            mr = m & 7
            o1 = pltpu.async_copy(out_v.at[0, pl.ds(0, NPIX)],
                                  out_hbm.at[:, m8, 0, mr], dsem)
            o2 = pltpu.async_copy(out_v.at[1, pl.ds(0, NPIX)],
                                  out_hbm.at[:, m8, 1, mr], dsem)
            o1.wait()
            o2.wait()

    return k(fm_flat, boxes_flat, bind, iofp, jofp)


def kernel(featuremap, boxes, box_ind):
    fm_flat = jnp.transpose(featuremap, (0, 2, 3, 1)).reshape(-1)
    boxes_flat = jnp.pad(boxes, ((0, MPAD - M), (0, 0))).reshape(-1)
    bind = jnp.pad(box_ind, (0, MPAD - M))
    p = jnp.arange(PADPIX, dtype=jnp.int32)
    pp = jnp.where(p < NPIX, p, 0)
    iofp = (pp // CW).astype(jnp.float32)
    jofp = (pp % CW).astype(jnp.float32)
    out = _roialign_sc(fm_flat, boxes_flat, bind, iofp, jofp)
    out = jnp.transpose(out, (1, 3, 2, 4, 0))
    return out.reshape(M, C, CH, CW)


# speedup vs baseline: 5.9481x; 1.5530x over previous
import functools

import jax
import jax.numpy as jnp
from jax import lax
from jax.experimental import pallas as pl
from jax.experimental.pallas import tpu as pltpu
from jax.experimental.pallas import tpu_sc as plsc

CH = 14
CW = 14
H = 64
W = 64
C = 256
PATCH = 13
PROW = PATCH * C
PSIZE = PATCH * PROW
NPIX = CH * CW
NCHUNK = 13
PADPIX = NCHUNK * 16
QC = C // 4

_NC = 2
_NS = 16
NTILES = _NC * _NS
M = 1000
MPAD = 1024
BPT = MPAD // NTILES


def _floor(x):
    t = x.astype(jnp.int32).astype(jnp.float32)
    return t - jnp.where(t > x, jnp.float32(1.0), jnp.float32(0.0))


def _ceil(x):
    t = x.astype(jnp.int32).astype(jnp.float32)
    return t + jnp.where(t < x, jnp.float32(1.0), jnp.float32(0.0))


def _roialign_sc(fm_flat, boxes_flat, bind, iofp, jofp):
    mesh = plsc.VectorSubcoreMesh(core_axis_name="c", subcore_axis_name="s",
                                  num_cores=_NC, num_subcores=_NS)

    @functools.partial(
        pl.kernel,
        out_type=jax.ShapeDtypeStruct((NPIX, M // 8, C // 128, 8, 128),
                                      jnp.float32),
        mesh=mesh,
        compiler_params=pltpu.CompilerParams(needs_layout_passes=False,
                                             use_tc_tiling_on_sc=False),
        scratch_types=[
            pltpu.VMEM((BPT * 4 + 16,), jnp.float32),
            pltpu.VMEM((BPT + 16,), jnp.int32),
            pltpu.VMEM((PADPIX,), jnp.float32),
            pltpu.VMEM((PADPIX,), jnp.float32),
            pltpu.VMEM((PADPIX,), jnp.float32),
            pltpu.VMEM((PADPIX,), jnp.float32),
            pltpu.VMEM((PADPIX,), jnp.float32),
            pltpu.VMEM((PADPIX,), jnp.float32),
            pltpu.VMEM((PADPIX,), jnp.int32),
            pltpu.VMEM((PADPIX,), jnp.int32),
            pltpu.VMEM((PADPIX,), jnp.int32),
            pltpu.VMEM((PADPIX,), jnp.int32),
            pltpu.VMEM((2 * PSIZE,), jnp.float32),
            pltpu.VMEM((PADPIX, QC), jnp.float32),
            pltpu.VMEM((PADPIX, QC), jnp.float32),
            pltpu.SemaphoreType.DMA,
            pltpu.SemaphoreType.DMA,
        ],
    )
    def k(fm_hbm, boxes_hbm, bind_hbm, iofp_hbm, jofp_hbm, out_hbm,
          boxes_v, bind_v, iofp_v, jofp_v,
          w1s, w2s, w3s, w4s, b1s, b2s, b3s, b4s,
          patch_v, stqa, stqb, psem, osem):
        wid = lax.axis_index("s") * _NC + lax.axis_index("c")
        m0 = wid * BPT
        count = jnp.minimum(BPT, M - m0)
        pltpu.sync_copy(boxes_hbm.at[pl.ds(pl.multiple_of(m0 * 4, BPT * 4),
                                           BPT * 4)],
                        boxes_v.at[pl.ds(0, BPT * 4)])
        pltpu.sync_copy(bind_hbm.at[pl.ds(pl.multiple_of(m0, BPT), BPT)],
                        bind_v.at[pl.ds(0, BPT)])
        pltpu.sync_copy(iofp_hbm, iofp_v)
        pltpu.sync_copy(jofp_hbm, jofp_v)
        iota = lax.broadcasted_iota(jnp.int32, (16,), 0)

        def _box_params(li):
            bv = boxes_v[pl.ds(4 * li, 16)]
            x1 = bv[0]
            y1 = bv[1]
            x2 = bv[2]
            y2 = bv[3]
            b = bind_v[pl.ds(li, 16)][0]
            spw = (x2 - x1) * jnp.float32(1.0 / CW)
            sph = (y2 - y1) * jnp.float32(1.0 / CH)
            nx0 = (x1 + spw * jnp.float32(0.5) - jnp.float32(0.5)) * jnp.float32(1.0 / (W - 1))
            ny0 = (y1 + sph * jnp.float32(0.5) - jnp.float32(0.5)) * jnp.float32(1.0 / (H - 1))
            nwd = spw * jnp.float32(CW - 1) * jnp.float32(1.0 / (W - 1))
            nht = sph * jnp.float32(CH - 1) * jnp.float32(1.0 / (H - 1))
            basex = nx0 * jnp.float32(W - 1)
            stepx = nwd * jnp.float32(W - 1) * jnp.float32(1.0 / (CW - 1))
            basey = ny0 * jnp.float32(H - 1)
            stepy = nht * jnp.float32(H - 1) * jnp.float32(1.0 / (CH - 1))
            x0 = jnp.clip(_floor(basex).astype(jnp.int32), 0, W - PATCH)
            y0 = jnp.clip(_floor(basey).astype(jnp.int32), 0, H - PATCH)
            return b, basex, stepx, basey, stepy, x0, y0

        def _issue_patch(li, buf):
            b, _, _, _, _, x0, y0 = _box_params(li)
            rowbase = ((b * H + y0) * W + x0) * C
            for dy in range(PATCH):
                pltpu.async_copy(
                    fm_hbm.at[pl.ds(pl.multiple_of(rowbase + dy * (W * C), C),
                                    PROW)],
                    patch_v.at[pl.ds(pl.multiple_of(buf * PSIZE, 256)
                                     + dy * PROW, PROW)],
                    psem)

        def _wait_patch():
            for _ in range(PATCH):
                pltpu.make_async_copy(
                    fm_hbm.at[pl.ds(0, PROW)],
                    patch_v.at[pl.ds(0, PROW)], psem).wait()

        def _drain_out(st):
            pltpu.make_async_copy(
                out_hbm.at[:, 0, 0, 0, pl.ds(0, QC)],
                st.at[pl.ds(0, NPIX)], osem).wait()

        _issue_patch(0, jnp.int32(0))

        @pl.loop(0, count)
        def _box(li):
            pb = li & 1
            pbase = pb * PSIZE
            _, basex, stepx, basey, stepy, x0, y0 = _box_params(li)

            _wait_patch()

            @pl.when(li + 1 < count)
            def _prefetch():
                _issue_patch(li + 1, (li + 1) & 1)

            @pl.loop(0, NCHUNK)
            def _prep(kk):
                iyf = iofp_v[pl.ds(kk * 16, 16)]
                jxf = jofp_v[pl.ds(kk * 16, 16)]
                in_y = basey + iyf * stepy
                in_x = basex + jxf * stepx
                vyv = jnp.where((in_y >= 0.0) & (in_y <= jnp.float32(H - 1)),
                                jnp.float32(1.0), jnp.float32(0.0))
                vxv = jnp.where((in_x >= 0.0) & (in_x <= jnp.float32(W - 1)),
                                jnp.float32(1.0), jnp.float32(0.0))
                ylo_f = _floor(in_y)
                xlo_f = _floor(in_x)
                yhi_f = _ceil(in_y)
                xhi_f = _ceil(in_x)
                ylerp = in_y - ylo_f
                xlerp = in_x - xlo_f
                pyl = jnp.clip(
                    jnp.clip(ylo_f, 0.0, jnp.float32(H - 1)).astype(jnp.int32)
                    - y0, 0, PATCH - 1)
                pyh = jnp.clip(
                    jnp.clip(yhi_f, 0.0, jnp.float32(H - 1)).astype(jnp.int32)
                    - y0, 0, PATCH - 1)
                pxl = jnp.clip(
                    jnp.clip(xlo_f, 0.0, jnp.float32(W - 1)).astype(jnp.int32)
                    - x0, 0, PATCH - 1)
                pxh = jnp.clip(
                    jnp.clip(xhi_f, 0.0, jnp.float32(W - 1)).astype(jnp.int32)
                    - x0, 0, PATCH - 1)
                vv = vyv * vxv
                omy = jnp.float32(1.0) - ylerp
                omx = jnp.float32(1.0) - xlerp
                w1s[pl.ds(kk * 16, 16)] = vv * omy * omx
                w2s[pl.ds(kk * 16, 16)] = vv * omy * xlerp
                w3s[pl.ds(kk * 16, 16)] = vv * ylerp * omx
                w4s[pl.ds(kk * 16, 16)] = vv * ylerp * xlerp
                b1s[pl.ds(kk * 16, 16)] = (pyl * PATCH + pxl) * C
                b2s[pl.ds(kk * 16, 16)] = (pyl * PATCH + pxh) * C
                b3s[pl.ds(kk * 16, 16)] = (pyh * PATCH + pxl) * C
                b4s[pl.ds(kk * 16, 16)] = (pyh * PATCH + pxh) * C

            m = m0 + li
            m8 = m >> 3
            mr = m & 7

            for q in range(4):
                st = stqa if q % 2 == 0 else stqb
                ct = q // 2
                colo = (q % 2) * QC
                if q >= 2:
                    _drain_out(st)
                else:
                    @pl.when(li > 0)
                    def _dr():
                        _drain_out(st)

                @pl.loop(0, NCHUNK)
                def _chunk(kk):
                    w1 = w1s[pl.ds(kk * 16, 16)]
                    w2 = w2s[pl.ds(kk * 16, 16)]
                    w3 = w3s[pl.ds(kk * 16, 16)]
                    w4 = w4s[pl.ds(kk * 16, 16)]
                    qoff = pbase + q * QC
                    b1 = b1s[pl.ds(kk * 16, 16)] + qoff
                    b2 = b2s[pl.ds(kk * 16, 16)] + qoff
                    b3 = b3s[pl.ds(kk * 16, 16)] + qoff
                    b4 = b4s[pl.ds(kk * 16, 16)] + qoff
                    pv = iota + kk * 16

                    @plsc.parallel_loop(0, QC, unroll=4,
                                        carry=(iota, b1, b2, b3, b4, pv,
                                               w1, w2, w3, w4))
                    def _chan(cc, carry):
                        cp, a1, a2, a3, a4, pv_, q1, q2, q3, q4 = carry
                        tl = plsc.load_gather(patch_v, [a1 + cp])
                        tr = plsc.load_gather(patch_v, [a2 + cp])
                        bl = plsc.load_gather(patch_v, [a3 + cp])
                        br = plsc.load_gather(patch_v, [a4 + cp])
                        acc = q1 * tl + q2 * tr + q3 * bl + q4 * br
                        plsc.store_scatter(st, [pv_, cp], acc)
                        cp2 = (cp + 1) & (QC - 1)
                        return (cp2, a1, a2, a3, a4, pv_, q1, q2, q3, q4)

                pltpu.async_copy(st.at[pl.ds(0, NPIX)],
                                 out_hbm.at[:, m8, ct, mr, pl.ds(colo, QC)],
                                 osem)

        _drain_out(stqa)
        _drain_out(stqb)

    return k(fm_flat, boxes_flat, bind, iofp, jofp)


def kernel(featuremap, boxes, box_ind):
    fm_flat = jnp.transpose(featuremap, (0, 2, 3, 1)).reshape(-1)
    boxes_flat = jnp.pad(boxes, ((0, MPAD - M), (0, 0))).reshape(-1)
    bind = jnp.pad(box_ind, (0, MPAD - M))
    p = jnp.arange(PADPIX, dtype=jnp.int32)
    pp = jnp.where(p < NPIX, p, 0)
    iofp = (pp // CW).astype(jnp.float32)
    jofp = (pp % CW).astype(jnp.float32)
    out = _roialign_sc(fm_flat, boxes_flat, bind, iofp, jofp)
    out = jnp.transpose(out, (1, 3, 2, 4, 0))
    return out.reshape(M, C, CH, CW)
